# baseline (speedup 1.0000x reference)
import functools

import jax
import jax.numpy as jnp
from jax import lax
from jax.experimental import pallas as pl
from jax.experimental.pallas import tpu as pltpu
from jax.experimental.pallas import tpu_sc as plsc

_N = 50000
_E = 800000
_IN_C = 109
_H1 = 320
_H2 = 64
_OUT = 32

_WC = 16
_NROW = 51200
_TILE_ROWS = _NROW // 16
_EB = 128
_KB = 200
_EPAD = 32 * _KB * _EB
_DUMP = _N

_BLK = 256
_GRID = _NROW // _BLK

_NC1 = 7
_NC2 = _H2 // _WC


@functools.lru_cache(maxsize=None)
def _build_mesh():
    return plsc.VectorSubcoreMesh(core_axis_name="c", subcore_axis_name="s",
                                  num_cores=2, num_subcores=16)


_NBUF = 4


def _sc_agg_multi(*refs):
    nc = len(refs) - 9 - _NBUF
    tables = refs[:nc]
    src_h, dst_h, zb_h, out_h, srcv, dstv = refs[nc:nc + 6]
    bufs = refs[nc + 6:nc + 6 + _NBUF]
    zb, acc, sem = refs[nc + 6 + _NBUF:]
    cid = lax.axis_index("c")
    sid = lax.axis_index("s")
    w = cid * 16 + sid
    pltpu.sync_copy(src_h.at[w], srcv)
    pltpu.sync_copy(dst_h.at[w], dstv)
    pltpu.sync_copy(zb_h, zb)
    base = sid * _TILE_ROWS

    for c in range(nc):
        def zloop(i, carry):
            pltpu.sync_copy(zb, acc.at[pl.ds(base + i * 128, 128)])
            return carry

        lax.fori_loop(0, _TILE_ROWS // 128, zloop, 0)
        plsc.subcore_barrier()

        for b in range(_NBUF):
            pltpu.async_copy(tables[c].at[srcv.at[b]], bufs[b], sem.at[b])

        def eloop(q, carry):
            j = q * _NBUF
            for b in range(_NBUF):
                jj = j + b
                pltpu.make_async_copy(tables[c].at[srcv.at[jj]],
                                      bufs[b], sem.at[b]).wait()
                pltpu.sync_copy(bufs[b], acc.at[dstv.at[jj]], add=True)

                @pl.when(jj + _NBUF < _KB)
                def _issue():
                    pltpu.async_copy(tables[c].at[srcv.at[jj + _NBUF]],
                                     bufs[b], sem.at[b])
            return carry

        lax.fori_loop(0, _KB // _NBUF, eloop, 0)
        plsc.subcore_barrier()
        pltpu.sync_copy(acc.at[pl.ds(base, _TILE_ROWS)],
                        out_h.at[c, cid, pl.ds(base, _TILE_ROWS)])
        plsc.subcore_barrier()


def _sc_deg(ones_h, zrow_h, dst_h, out_h, dstv, onesv, zv, acc):
    cid = lax.axis_index("c")
    sid = lax.axis_index("s")
    w = cid * 16 + sid
    pltpu.sync_copy(dst_h.at[w], dstv)
    pltpu.sync_copy(ones_h, onesv)
    pltpu.sync_copy(zrow_h, zv)
    base = sid * _TILE_ROWS
    pltpu.sync_copy(zv, acc.at[pl.ds(base, _TILE_ROWS)])
    plsc.subcore_barrier()

    def eloop(j, carry):
        pltpu.sync_copy(onesv, acc.at[dstv.at[j]], add=True)
        return carry

    lax.fori_loop(0, _KB, eloop, 0)
    plsc.subcore_barrier()
    pltpu.sync_copy(acc.at[pl.ds(base, _TILE_ROWS)],
                    out_h.at[cid, pl.ds(base, _TILE_ROWS)])


@functools.lru_cache(maxsize=None)
def _build_agg(nc):
    return pl.kernel(
        _sc_agg_multi,
        out_type=jax.ShapeDtypeStruct((nc, 2, _NROW, _WC), jnp.float32),
        mesh=_build_mesh(),
        scratch_types=[
            pltpu.VMEM((_KB, _EB), jnp.int32),
            pltpu.VMEM((_KB, _EB), jnp.int32),
        ] + [pltpu.VMEM((_EB, _WC), jnp.float32)] * _NBUF + [
            pltpu.VMEM((128, _WC), jnp.float32),
            pltpu.VMEM_SHARED((_NROW, _WC), jnp.float32),
            pltpu.SemaphoreType.DMA((_NBUF,)),
        ],
        compiler_params=pltpu.CompilerParams(use_tc_tiling_on_sc=False),
    )


@functools.lru_cache(maxsize=None)
def _build_deg():
    return pl.kernel(
        _sc_deg,
        out_type=jax.ShapeDtypeStruct((2, _NROW), jnp.float32),
        mesh=_build_mesh(),
        scratch_types=[
            pltpu.VMEM((_KB, _EB), jnp.int32),
            pltpu.VMEM((_EB,), jnp.float32),
            pltpu.VMEM((_TILE_ROWS,), jnp.float32),
            pltpu.VMEM_SHARED((_NROW,), jnp.float32),
        ],
        compiler_params=pltpu.CompilerParams(use_tc_tiling_on_sc=False),
    )


def _deg_partials(dst3):
    ones = jnp.ones((_EB,), jnp.float32)
    zrow = jnp.zeros((_TILE_ROWS,), jnp.float32)
    return _build_deg()(ones, zrow, dst3)


def _agg_edges(tables, src3, dst3):
    zb = jnp.zeros((128, _WC), jnp.float32)
    out = _build_agg(len(tables))(*tables, src3, dst3, zb)
    return [out[c] for c in range(len(tables))]


def _tc1_body(degp_ref, x_ref, dinv_ref, *g_refs):
    d = degp_ref[:, 0:1] + degp_ref[:, 1:2] + 1.0
    dv = lax.rsqrt(d)
    dinv_ref[...] = dv
    g = x_ref[...] * dv
    for k, gr in enumerate(g_refs):
        gr[...] = g[:, k * _WC:(k + 1) * _WC]


def _tc1(degp, x_pad):
    return pl.pallas_call(
        _tc1_body,
        grid=(_GRID,),
        in_specs=[
            pl.BlockSpec((_BLK, 2), lambda i: (i, 0)),
            pl.BlockSpec((_BLK, _NC1 * _WC), lambda i: (i, 0)),
        ],
        out_specs=[pl.BlockSpec((_BLK, 1), lambda i: (i, 0))]
        + [pl.BlockSpec((_BLK, _WC), lambda i: (i, 0))] * _NC1,
        out_shape=[jax.ShapeDtypeStruct((_NROW, 1), jnp.float32)]
        + [jax.ShapeDtypeStruct((_NROW, _WC), jnp.float32)] * _NC1,
    )(degp, x_pad)


def _tc2_body(*refs):
    ps = refs[:_NC1]
    gcs = refs[_NC1:2 * _NC1]
    dinv, w1_ref, b1_ref, w2_ref, g2a_ref, g2b_ref, g2c_ref, g2d_ref = refs[2 * _NC1:]
    dv = dinv[...]
    acc = None
    for k, (p, gc) in enumerate(zip(ps, gcs)):
        t = (p[0] + p[1] + gc[...]) * dv
        part = jnp.dot(t, w1_ref[k * _WC:(k + 1) * _WC, :],
                       preferred_element_type=jnp.float32)
        acc = part if acc is None else acc + part
    h1 = jnp.maximum(acc + b1_ref[...], 0.0)
    m2 = jnp.dot(h1, w2_ref[...], preferred_element_type=jnp.float32)
    g2 = m2 * dv
    for k, o in enumerate((g2a_ref, g2b_ref, g2c_ref, g2d_ref)):
        o[...] = g2[:, k * _WC:(k + 1) * _WC]


def _tc2(parts, gcs, dinv, w1p, b1, w2):
    pspec = pl.BlockSpec((2, _BLK, _WC), lambda i: (0, i, 0))
    cspec = pl.BlockSpec((_BLK, _WC), lambda i: (i, 0))
    return pl.pallas_call(
        _tc2_body,
        grid=(_GRID,),
        in_specs=[pspec] * _NC1 + [cspec] * _NC1 + [
            pl.BlockSpec((_BLK, 1), lambda i: (i, 0)),
            pl.BlockSpec((_NC1 * _WC, _H1), lambda i: (0, 0)),
            pl.BlockSpec((1, _H1), lambda i: (0, 0)),
            pl.BlockSpec((_H1, _H2), lambda i: (0, 0)),
        ],
        out_specs=[cspec] * _NC2,
        out_shape=[jax.ShapeDtypeStruct((_NROW, _WC), jnp.float32)] * _NC2,
    )(*parts, *gcs, dinv, w1p, b1, w2)


def _tc3_body(*refs):
    qs = refs[:_NC2]
    gcs = refs[_NC2:2 * _NC2]
    dinv, b2_ref = refs[2 * _NC2:2 * _NC2 + 2]
    outs = refs[2 * _NC2 + 2:]
    dv = dinv[...]
    for k, (q, gc, o) in enumerate(zip(qs, gcs, outs)):
        pre = (q[0] + q[1] + gc[...]) * dv
        h2 = jnp.maximum(pre + b2_ref[:, k * _WC:(k + 1) * _WC], 0.0)
        o[...] = h2 * dv


def _tc3(parts, gcs, dinv, b2):
    pspec = pl.BlockSpec((2, _BLK, _WC), lambda i: (0, i, 0))
    cspec = pl.BlockSpec((_BLK, _WC), lambda i: (i, 0))
    return pl.pallas_call(
        _tc3_body,
        grid=(_GRID,),
        in_specs=[pspec] * _NC2 + [cspec] * _NC2 + [
            pl.BlockSpec((_BLK, 1), lambda i: (i, 0)),
            pl.BlockSpec((1, _H2), lambda i: (0, 0)),
        ],
        out_specs=[cspec] * _NC2,
        out_shape=[jax.ShapeDtypeStruct((_NROW, _WC), jnp.float32)] * _NC2,
    )(*parts, *gcs, dinv, b2)


def _tc4_body(*refs):
    rs = refs[:_NC2]
    gcs = refs[_NC2:2 * _NC2]
    dinv, wmu_ref, bmu_ref, wls_ref, bls_ref, mu_ref, ls_ref = refs[2 * _NC2:]
    dv = dinv[...]
    mu = bmu_ref[...]
    ls = bls_ref[...]
    for k, (r, gc) in enumerate(zip(rs, gcs)):
        a = (r[0] + r[1] + gc[...]) * dv
        mu = mu + jnp.dot(a, wmu_ref[k * _WC:(k + 1) * _WC, :],
                          preferred_element_type=jnp.float32)
        ls = ls + jnp.dot(a, wls_ref[k * _WC:(k + 1) * _WC, :],
                          preferred_element_type=jnp.float32)
    mu_ref[...] = mu
    ls_ref[...] = ls


def _tc4(parts, gcs, dinv, wmu, bmu, wls, bls):
    pspec = pl.BlockSpec((2, _BLK, _WC), lambda i: (0, i, 0))
    cspec = pl.BlockSpec((_BLK, _WC), lambda i: (i, 0))
    ospec = pl.BlockSpec((_BLK, _OUT), lambda i: (i, 0))
    wspec = pl.BlockSpec((_H2, _OUT), lambda i: (0, 0))
    bspec = pl.BlockSpec((1, _OUT), lambda i: (0, 0))
    return pl.pallas_call(
        _tc4_body,
        grid=(_GRID,),
        in_specs=[pspec] * _NC2 + [cspec] * _NC2 + [
            pl.BlockSpec((_BLK, 1), lambda i: (i, 0)),
            wspec, bspec, wspec, bspec,
        ],
        out_specs=[ospec, ospec],
        out_shape=[jax.ShapeDtypeStruct((_NROW, _OUT), jnp.float32)] * 2,
    )(*parts, *gcs, dinv, wmu, bmu, wls, bls)


def kernel(x, edge_index, W1, b1, W2, b2, W_mu, b_mu, W_ls, b_ls):
    src = edge_index[0].astype(jnp.int32)
    dst = edge_index[1].astype(jnp.int32)
    pad = _EPAD - _E
    src3 = jnp.concatenate([src, jnp.zeros((pad,), jnp.int32)]).reshape(32, _KB, _EB)
    dst3 = jnp.concatenate([dst, jnp.full((pad,), _DUMP, jnp.int32)]).reshape(32, _KB, _EB)

    ncol = _NC1 * _WC
    x_pad = jnp.zeros((_NROW, ncol), jnp.float32).at[:_N, :_IN_C].set(x)
    w1p = jnp.zeros((ncol, _H1), jnp.float32).at[:_IN_C, :].set(W1)
    b1r = b1.reshape(1, _H1)
    b2r = b2.reshape(1, _H2)
    bmur = b_mu.reshape(1, _OUT)
    blsr = b_ls.reshape(1, _OUT)

    degp = jnp.transpose(_deg_partials(dst3))
    tc1_out = _tc1(degp, x_pad)
    dinv, gcs1 = tc1_out[0], tuple(tc1_out[1:])

    parts1 = _agg_edges(gcs1, src3, dst3)
    gcs2 = tuple(_tc2(parts1, gcs1, dinv, w1p, b1r, W2))

    parts2 = _agg_edges(gcs2, src3, dst3)
    gcs3 = tuple(_tc3(parts2, gcs2, dinv, b2r))

    parts3 = _agg_edges(gcs3, src3, dst3)
    mu, ls = _tc4(parts3, gcs3, dinv, W_mu, bmur, W_ls, blsr)

    return (mu[:_N], ls[:_N])


# speedup vs baseline: 1.0087x; 1.0087x over previous
import functools

import jax
import jax.numpy as jnp
from jax import lax
from jax.experimental import pallas as pl
from jax.experimental.pallas import tpu as pltpu
from jax.experimental.pallas import tpu_sc as plsc

_N = 50000
_E = 800000
_IN_C = 109
_H1 = 320
_H2 = 64
_OUT = 32

_WC = 16
_NROW = 51200
_TILE_ROWS = _NROW // 16
_EB = 256
_KB = 100
_EPAD = 32 * _KB * _EB
_DUMP = _N

_BLK = 256
_GRID = _NROW // _BLK

_NC1 = 7
_NC2 = _H2 // _WC


@functools.lru_cache(maxsize=None)
def _build_mesh():
    return plsc.VectorSubcoreMesh(core_axis_name="c", subcore_axis_name="s",
                                  num_cores=2, num_subcores=16)


_NBUF = 4


def _sc_agg_multi(*refs):
    nc = len(refs) - 9 - _NBUF
    tables = refs[:nc]
    src_h, dst_h, zb_h, out_h, srcv, dstv = refs[nc:nc + 6]
    bufs = refs[nc + 6:nc + 6 + _NBUF]
    zb, acc, sem = refs[nc + 6 + _NBUF:]
    cid = lax.axis_index("c")
    sid = lax.axis_index("s")
    w = cid * 16 + sid
    pltpu.sync_copy(src_h.at[w], srcv)
    pltpu.sync_copy(dst_h.at[w], dstv)
    pltpu.sync_copy(zb_h, zb)
    base = sid * _TILE_ROWS

    for c in range(nc):
        def zloop(i, carry):
            pltpu.sync_copy(zb, acc.at[pl.ds(base + i * 128, 128)])
            return carry

        lax.fori_loop(0, _TILE_ROWS // 128, zloop, 0)
        plsc.subcore_barrier()

        for b in range(_NBUF):
            pltpu.async_copy(tables[c].at[srcv.at[b]], bufs[b], sem.at[b])

        def eloop(q, carry):
            j = q * _NBUF
            for b in range(_NBUF):
                jj = j + b
                pltpu.make_async_copy(tables[c].at[srcv.at[jj]],
                                      bufs[b], sem.at[b]).wait()
                pltpu.sync_copy(bufs[b], acc.at[dstv.at[jj]], add=True)

                @pl.when(jj + _NBUF < _KB)
                def _issue():
                    pltpu.async_copy(tables[c].at[srcv.at[jj + _NBUF]],
                                     bufs[b], sem.at[b])
            return carry

        lax.fori_loop(0, _KB // _NBUF, eloop, 0)
        plsc.subcore_barrier()
        pltpu.sync_copy(acc.at[pl.ds(base, _TILE_ROWS)],
                        out_h.at[c, cid, pl.ds(base, _TILE_ROWS)])
        plsc.subcore_barrier()


def _sc_deg(ones_h, zrow_h, dst_h, out_h, dstv, onesv, zv, acc):
    cid = lax.axis_index("c")
    sid = lax.axis_index("s")
    w = cid * 16 + sid
    pltpu.sync_copy(dst_h.at[w], dstv)
    pltpu.sync_copy(ones_h, onesv)
    pltpu.sync_copy(zrow_h, zv)
    base = sid * _TILE_ROWS
    pltpu.sync_copy(zv, acc.at[pl.ds(base, _TILE_ROWS)])
    plsc.subcore_barrier()

    def eloop(j, carry):
        pltpu.sync_copy(onesv, acc.at[dstv.at[j]], add=True)
        return carry

    lax.fori_loop(0, _KB, eloop, 0)
    plsc.subcore_barrier()
    pltpu.sync_copy(acc.at[pl.ds(base, _TILE_ROWS)],
                    out_h.at[cid, pl.ds(base, _TILE_ROWS)])


@functools.lru_cache(maxsize=None)
def _build_agg(nc):
    return pl.kernel(
        _sc_agg_multi,
        out_type=jax.ShapeDtypeStruct((nc, 2, _NROW, _WC), jnp.float32),
        mesh=_build_mesh(),
        scratch_types=[
            pltpu.VMEM((_KB, _EB), jnp.int32),
            pltpu.VMEM((_KB, _EB), jnp.int32),
        ] + [pltpu.VMEM((_EB, _WC), jnp.float32)] * _NBUF + [
            pltpu.VMEM((128, _WC), jnp.float32),
            pltpu.VMEM_SHARED((_NROW, _WC), jnp.float32),
            pltpu.SemaphoreType.DMA((_NBUF,)),
        ],
        compiler_params=pltpu.CompilerParams(use_tc_tiling_on_sc=False),
    )


@functools.lru_cache(maxsize=None)
def _build_deg():
    return pl.kernel(
        _sc_deg,
        out_type=jax.ShapeDtypeStruct((2, _NROW), jnp.float32),
        mesh=_build_mesh(),
        scratch_types=[
            pltpu.VMEM((_KB, _EB), jnp.int32),
            pltpu.VMEM((_EB,), jnp.float32),
            pltpu.VMEM((_TILE_ROWS,), jnp.float32),
            pltpu.VMEM_SHARED((_NROW,), jnp.float32),
        ],
        compiler_params=pltpu.CompilerParams(use_tc_tiling_on_sc=False),
    )


def _deg_partials(dst3):
    ones = jnp.ones((_EB,), jnp.float32)
    zrow = jnp.zeros((_TILE_ROWS,), jnp.float32)
    return _build_deg()(ones, zrow, dst3)


def _agg_edges(tables, src3, dst3):
    zb = jnp.zeros((128, _WC), jnp.float32)
    out = _build_agg(len(tables))(*tables, src3, dst3, zb)
    return [out[c] for c in range(len(tables))]


def _tc1_body(degp_ref, x_ref, dinv_ref, *g_refs):
    d = degp_ref[:, 0:1] + degp_ref[:, 1:2] + 1.0
    dv = lax.rsqrt(d)
    dinv_ref[...] = dv
    g = x_ref[...] * dv
    for k, gr in enumerate(g_refs):
        gr[...] = g[:, k * _WC:(k + 1) * _WC]


def _tc1(degp, x_pad):
    return pl.pallas_call(
        _tc1_body,
        grid=(_GRID,),
        in_specs=[
            pl.BlockSpec((_BLK, 2), lambda i: (i, 0)),
            pl.BlockSpec((_BLK, _NC1 * _WC), lambda i: (i, 0)),
        ],
        out_specs=[pl.BlockSpec((_BLK, 1), lambda i: (i, 0))]
        + [pl.BlockSpec((_BLK, _WC), lambda i: (i, 0))] * _NC1,
        out_shape=[jax.ShapeDtypeStruct((_NROW, 1), jnp.float32)]
        + [jax.ShapeDtypeStruct((_NROW, _WC), jnp.float32)] * _NC1,
    )(degp, x_pad)


def _tc2_body(*refs):
    ps = refs[:_NC1]
    gcs = refs[_NC1:2 * _NC1]
    dinv, w1_ref, b1_ref, w2_ref, g2a_ref, g2b_ref, g2c_ref, g2d_ref = refs[2 * _NC1:]
    dv = dinv[...]
    acc = None
    for k, (p, gc) in enumerate(zip(ps, gcs)):
        t = (p[0] + p[1] + gc[...]) * dv
        part = jnp.dot(t, w1_ref[k * _WC:(k + 1) * _WC, :],
                       preferred_element_type=jnp.float32)
        acc = part if acc is None else acc + part
    h1 = jnp.maximum(acc + b1_ref[...], 0.0)
    m2 = jnp.dot(h1, w2_ref[...], preferred_element_type=jnp.float32)
    g2 = m2 * dv
    for k, o in enumerate((g2a_ref, g2b_ref, g2c_ref, g2d_ref)):
        o[...] = g2[:, k * _WC:(k + 1) * _WC]


def _tc2(parts, gcs, dinv, w1p, b1, w2):
    pspec = pl.BlockSpec((2, _BLK, _WC), lambda i: (0, i, 0))
    cspec = pl.BlockSpec((_BLK, _WC), lambda i: (i, 0))
    return pl.pallas_call(
        _tc2_body,
        grid=(_GRID,),
        in_specs=[pspec] * _NC1 + [cspec] * _NC1 + [
            pl.BlockSpec((_BLK, 1), lambda i: (i, 0)),
            pl.BlockSpec((_NC1 * _WC, _H1), lambda i: (0, 0)),
            pl.BlockSpec((1, _H1), lambda i: (0, 0)),
            pl.BlockSpec((_H1, _H2), lambda i: (0, 0)),
        ],
        out_specs=[cspec] * _NC2,
        out_shape=[jax.ShapeDtypeStruct((_NROW, _WC), jnp.float32)] * _NC2,
    )(*parts, *gcs, dinv, w1p, b1, w2)


def _tc3_body(*refs):
    qs = refs[:_NC2]
    gcs = refs[_NC2:2 * _NC2]
    dinv, b2_ref = refs[2 * _NC2:2 * _NC2 + 2]
    outs = refs[2 * _NC2 + 2:]
    dv = dinv[...]
    for k, (q, gc, o) in enumerate(zip(qs, gcs, outs)):
        pre = (q[0] + q[1] + gc[...]) * dv
        h2 = jnp.maximum(pre + b2_ref[:, k * _WC:(k + 1) * _WC], 0.0)
        o[...] = h2 * dv


def _tc3(parts, gcs, dinv, b2):
    pspec = pl.BlockSpec((2, _BLK, _WC), lambda i: (0, i, 0))
    cspec = pl.BlockSpec((_BLK, _WC), lambda i: (i, 0))
    return pl.pallas_call(
        _tc3_body,
        grid=(_GRID,),
        in_specs=[pspec] * _NC2 + [cspec] * _NC2 + [
            pl.BlockSpec((_BLK, 1), lambda i: (i, 0)),
            pl.BlockSpec((1, _H2), lambda i: (0, 0)),
        ],
        out_specs=[cspec] * _NC2,
        out_shape=[jax.ShapeDtypeStruct((_NROW, _WC), jnp.float32)] * _NC2,
    )(*parts, *gcs, dinv, b2)


def _tc4_body(*refs):
    rs = refs[:_NC2]
    gcs = refs[_NC2:2 * _NC2]
    dinv, wmu_ref, bmu_ref, wls_ref, bls_ref, mu_ref, ls_ref = refs[2 * _NC2:]
    dv = dinv[...]
    mu = bmu_ref[...]
    ls = bls_ref[...]
    for k, (r, gc) in enumerate(zip(rs, gcs)):
        a = (r[0] + r[1] + gc[...]) * dv
        mu = mu + jnp.dot(a, wmu_ref[k * _WC:(k + 1) * _WC, :],
                          preferred_element_type=jnp.float32)
        ls = ls + jnp.dot(a, wls_ref[k * _WC:(k + 1) * _WC, :],
                          preferred_element_type=jnp.float32)
    mu_ref[...] = mu
    ls_ref[...] = ls


def _tc4(parts, gcs, dinv, wmu, bmu, wls, bls):
    pspec = pl.BlockSpec((2, _BLK, _WC), lambda i: (0, i, 0))
    cspec = pl.BlockSpec((_BLK, _WC), lambda i: (i, 0))
    ospec = pl.BlockSpec((_BLK, _OUT), lambda i: (i, 0))
    wspec = pl.BlockSpec((_H2, _OUT), lambda i: (0, 0))
    bspec = pl.BlockSpec((1, _OUT), lambda i: (0, 0))
    return pl.pallas_call(
        _tc4_body,
        grid=(_GRID,),
        in_specs=[pspec] * _NC2 + [cspec] * _NC2 + [
            pl.BlockSpec((_BLK, 1), lambda i: (i, 0)),
            wspec, bspec, wspec, bspec,
        ],
        out_specs=[ospec, ospec],
        out_shape=[jax.ShapeDtypeStruct((_NROW, _OUT), jnp.float32)] * 2,
    )(*parts, *gcs, dinv, wmu, bmu, wls, bls)


def kernel(x, edge_index, W1, b1, W2, b2, W_mu, b_mu, W_ls, b_ls):
    src = edge_index[0].astype(jnp.int32)
    dst = edge_index[1].astype(jnp.int32)
    pad = _EPAD - _E
    src3 = jnp.concatenate([src, jnp.zeros((pad,), jnp.int32)]).reshape(32, _KB, _EB)
    dst3 = jnp.concatenate([dst, jnp.full((pad,), _DUMP, jnp.int32)]).reshape(32, _KB, _EB)

    ncol = _NC1 * _WC
    x_pad = jnp.zeros((_NROW, ncol), jnp.float32).at[:_N, :_IN_C].set(x)
    w1p = jnp.zeros((ncol, _H1), jnp.float32).at[:_IN_C, :].set(W1)
    b1r = b1.reshape(1, _H1)
    b2r = b2.reshape(1, _H2)
    bmur = b_mu.reshape(1, _OUT)
    blsr = b_ls.reshape(1, _OUT)

    degp = jnp.transpose(_deg_partials(dst3))
    tc1_out = _tc1(degp, x_pad)
    dinv, gcs1 = tc1_out[0], tuple(tc1_out[1:])

    parts1 = _agg_edges(gcs1, src3, dst3)
    gcs2 = tuple(_tc2(parts1, gcs1, dinv, w1p, b1r, W2))

    parts2 = _agg_edges(gcs2, src3, dst3)
    gcs3 = tuple(_tc3(parts2, gcs2, dinv, b2r))

    parts3 = _agg_edges(gcs3, src3, dst3)
    mu, ls = _tc4(parts3, gcs3, dinv, W_mu, bmur, W_ls, blsr)

    return (mu[:_N], ls[:_N])
